# baseline (speedup 1.0000x reference)
import functools

import jax
import jax.numpy as jnp
from jax import lax
from jax.experimental import pallas as pl
from jax.experimental.pallas import tpu as pltpu
from jax.experimental.pallas import tpu_sc as plsc

N = 10000
E = 320000
D = 128
NC = 2
NS = 16
RPT = 624
TAIL = N - NS * RPT
CHUNK = 96
NCHUNK = -(-(E // NS) // CHUNK)
EPT = NCHUNK * CHUNK
EPAD = NS * EPT
NPAD = N + 16
BM = 2000


def _mmT(a, w):
    return lax.dot_general(a, w, (((1,), (1,)), ((), ())),
                           preferred_element_type=jnp.float32)


def _proj_body(x_ref, wp_ref, bp_ref, wn_ref, ws_ref, bs_ref, y_ref, z_ref):
    x = x_ref[0]
    h = jnp.maximum(_mmT(x, wp_ref[0]) + bp_ref[0, 0], 0.0)
    h = jnp.where(jnp.isnan(h), 0.0, h)
    y_ref[0] = _mmT(h, wn_ref[0])
    z_ref[0] = _mmT(h, ws_ref[0]) + bs_ref[0, 0]


def _mid_body(z_ref, s_ref, c_ref, wn_ref, ws_ref, bs_ref, y_ref, z2_ref):
    cnt = jnp.maximum(c_ref[0][:, 0:1], 1.0)
    h = jnp.maximum(z_ref[0] + s_ref[0] / cnt, 0.0)
    y_ref[0] = _mmT(h, wn_ref[0])
    z2_ref[0] = _mmT(h, ws_ref[0]) + bs_ref[0, 0]


def _fin_body(z_ref, s_ref, c_ref, o_ref):
    cnt = jnp.maximum(c_ref[0][:, 0:1], 1.0)
    o_ref[0] = jnp.maximum(z_ref[0] + s_ref[0] / cnt, 0.0)


def _wspec():
    return pl.BlockSpec((1, D, D), lambda t, r: (t, 0, 0))


def _bspec():
    return pl.BlockSpec((1, 1, D), lambda t, r: (t, 0, 0))


def _nspec():
    return pl.BlockSpec((1, BM, D), lambda t, r: (t, r, 0))


def _stage_proj(x_all, wp, bp, wn, ws, bs):
    return pl.pallas_call(
        _proj_body,
        grid=(2, N // BM),
        in_specs=[_nspec(), _wspec(), _bspec(), _wspec(), _wspec(), _bspec()],
        out_specs=[_nspec(), _nspec()],
        out_shape=[jax.ShapeDtypeStruct((2, N, D), jnp.float32)] * 2,
    )(x_all, wp, bp, wn, ws, bs)


def _stage_mid(z1, s1, c1, wn, ws, bs):
    return pl.pallas_call(
        _mid_body,
        grid=(2, N // BM),
        in_specs=[_nspec(), _nspec(), _nspec(), _wspec(), _wspec(), _bspec()],
        out_specs=[_nspec(), _nspec()],
        out_shape=[jax.ShapeDtypeStruct((2, N, D), jnp.float32)] * 2,
    )(z1, s1, c1, wn, ws, bs)


def _stage_fin(z2, s2, c2):
    return pl.pallas_call(
        _fin_body,
        grid=(2, N // BM),
        in_specs=[_nspec(), _nspec(), _nspec()],
        out_specs=_nspec(),
        out_shape=jax.ShapeDtypeStruct((2, N, D), jnp.float32),
    )(z2, s2, c2)


def _sc_mesh():
    return plsc.VectorSubcoreMesh(core_axis_name="c", subcore_axis_name="s",
                                  num_cores=NC, num_subcores=NS)


def _zero_phase(zf_hbm, acc, s):
    pltpu.sync_copy(zf_hbm, acc.at[pl.ds(s * RPT, RPT)])

    @pl.when(s == NS - 1)
    def _zero_tail():
        pltpu.sync_copy(zf_hbm.at[pl.ds(0, TAIL)],
                        acc.at[pl.ds(NS * RPT, TAIL)])


def _out_phase(acc, out, c, s):
    dts = 1 - c
    pltpu.sync_copy(acc.at[pl.ds(s * RPT, RPT)],
                    out.at[dts, pl.ds(s * RPT, RPT)])

    @pl.when(s == NS - 1)
    def _out_tail():
        pltpu.sync_copy(acc.at[pl.ds(NS * RPT, TAIL)],
                        out.at[dts, pl.ds(NS * RPT, TAIL)])


NBUF = 4


def _sc_segsum(y_flat, idx_packed, zfeat):
    scratch = (
        [pltpu.VMEM((2, CHUNK), jnp.int32)] * NBUF
        + [pltpu.VMEM((CHUNK, D), jnp.float32)] * NBUF
        + [pltpu.VMEM_SHARED((NPAD, D), jnp.float32)]
        + [pltpu.SemaphoreType.DMA] * (3 * NBUF)
    )

    @functools.partial(
        pl.kernel,
        out_type=jax.ShapeDtypeStruct((NC, N, D), jnp.float32),
        mesh=_sc_mesh(),
        scratch_types=scratch,
    )
    def body(y_hbm, idx_hbm, zf_hbm, s_out, *scr):
        ibuf = scr[0:NBUF]
        rows = scr[NBUF:2 * NBUF]
        acc = scr[2 * NBUF]
        isem = scr[2 * NBUF + 1:][0:NBUF]
        gsem = scr[2 * NBUF + 1:][NBUF:2 * NBUF]
        ssem = scr[2 * NBUF + 1:][2 * NBUF:3 * NBUF]

        c = lax.axis_index("c")
        s = lax.axis_index("s")
        _zero_phase(zf_hbm, acc, s)
        plsc.subcore_barrier()

        cbase = (c * NS + s) * NCHUNK

        def fire_idx(j, b):
            pltpu.async_copy(idx_hbm.at[cbase + j], ibuf[b], isem[b])

        def fire_gather(b):
            pltpu.make_async_copy(idx_hbm.at[cbase], ibuf[b], isem[b]).wait()
            pltpu.async_copy(y_hbm.at[ibuf[b].at[0]], rows[b], gsem[b])

        def fire_scatter(b):
            pltpu.make_async_copy(y_hbm.at[ibuf[b].at[0]], rows[b],
                                  gsem[b]).wait()
            pltpu.async_copy(rows[b], acc.at[ibuf[b].at[1]], ssem[b],
                             add=True)

        def wait_scatter(b):
            pltpu.make_async_copy(rows[b], acc.at[ibuf[b].at[1]],
                                  ssem[b]).wait()

        def macro(k, carry):
            for u in range(NBUF):
                t = k * NBUF + u
                pl.when((t >= NBUF) & (t - NBUF < NCHUNK))(
                    lambda b=u: wait_scatter(b))
                pl.when(t < NCHUNK)(lambda b=u, tt=t: fire_idx(tt, b))
                pl.when((t >= 1) & (t - 1 < NCHUNK))(
                    lambda b=(u - 1) % NBUF: fire_gather(b))
                pl.when((t >= 3) & (t - 3 < NCHUNK))(
                    lambda b=(u - 3) % NBUF: fire_scatter(b))
            return carry

        lax.fori_loop(0, (NCHUNK + NBUF - 1) // NBUF + 1, macro, 0)
        plsc.subcore_barrier()
        _out_phase(acc, s_out, c, s)

    return body(y_flat, idx_packed, zfeat)


def _sc_count(idx_packed, zfeat):
    @functools.partial(
        pl.kernel,
        out_type=jax.ShapeDtypeStruct((NC, N, D), jnp.float32),
        mesh=_sc_mesh(),
        scratch_types=(
            [pltpu.VMEM((2, CHUNK), jnp.int32)] * NBUF
            + [pltpu.VMEM((CHUNK, D), jnp.float32),
               pltpu.VMEM_SHARED((NPAD, D), jnp.float32)]
            + [pltpu.SemaphoreType.DMA] * (2 * NBUF)
        ),
    )
    def body(idx_hbm, zf_hbm, c_out, *scr):
        ibuf = scr[0:NBUF]
        ones = scr[NBUF]
        acc = scr[NBUF + 1]
        isem = scr[NBUF + 2:][0:NBUF]
        ssem = scr[NBUF + 2:][NBUF:2 * NBUF]

        c = lax.axis_index("c")
        s = lax.axis_index("s")
        _zero_phase(zf_hbm, acc, s)
        one16 = jnp.full((16,), 1.0, jnp.float32)

        def fill(i, carry):
            for g in range(D // 16):
                ones[i, pl.ds(g * 16, 16)] = one16
            return carry

        lax.fori_loop(0, CHUNK, fill, 0)
        plsc.subcore_barrier()

        cbase = (c * NS + s) * NCHUNK

        def fire_idx(j, b):
            pltpu.async_copy(idx_hbm.at[cbase + j], ibuf[b], isem[b])

        def fire_scatter(b):
            pltpu.make_async_copy(idx_hbm.at[cbase], ibuf[b], isem[b]).wait()
            pltpu.async_copy(ones, acc.at[ibuf[b].at[1]], ssem[b], add=True)

        def wait_scatter(b):
            pltpu.make_async_copy(ones, acc.at[ibuf[b].at[1]],
                                  ssem[b]).wait()

        def macro(k, carry):
            for u in range(NBUF):
                t = k * NBUF + u
                pl.when((t >= NBUF) & (t - NBUF < NCHUNK))(
                    lambda b=u: wait_scatter(b))
                pl.when(t < NCHUNK)(lambda b=u, tt=t: fire_idx(tt, b))
                pl.when((t >= 1) & (t - 1 < NCHUNK))(
                    lambda b=(u - 1) % NBUF: fire_scatter(b))
            return carry

        lax.fori_loop(0, (NCHUNK + NBUF - 1) // NBUF + 1, macro, 0)
        plsc.subcore_barrier()
        _out_phase(acc, c_out, c, s)

    return body(idx_packed, zfeat)


def kernel(x_user, x_item, edge_index_user_to_item, edge_index_item_rev_user,
           params):
    f32 = jnp.float32
    p = params
    x_all = jnp.stack([x_user, x_item]).astype(f32)

    wp = jnp.stack([p["proj"]["user"]["W"], p["proj"]["item"]["W"]])
    bp = jnp.stack([p["proj"]["user"]["b"], p["proj"]["item"]["b"]])[:, None, :]
    c1u = p["conv1"]["user__to__item"]
    c1i = p["conv1"]["item__rev__user"]
    c2u = p["conv2"]["user__to__item"]
    c2i = p["conv2"]["item__rev__user"]
    wn1 = jnp.stack([c1u["Wn"], c1i["Wn"]])
    ws1 = jnp.stack([c1i["Ws"], c1u["Ws"]])
    bs1 = jnp.stack([c1i["bs"], c1u["bs"]])[:, None, :]
    wn2 = jnp.stack([c2u["Wn"], c2i["Wn"]])
    ws2 = jnp.stack([c2i["Ws"], c2u["Ws"]])
    bs2 = jnp.stack([c2i["bs"], c2u["bs"]])[:, None, :]

    src_flat = jnp.concatenate([edge_index_user_to_item[0].astype(jnp.int32),
                                edge_index_item_rev_user[0].astype(jnp.int32)
                                + N])
    dst_flat = jnp.concatenate([edge_index_user_to_item[1].astype(jnp.int32),
                                edge_index_item_rev_user[1].astype(jnp.int32)])
    pad = EPAD - E
    src_p = jnp.concatenate(
        [src_flat[:E], jnp.zeros((pad,), jnp.int32),
         src_flat[E:], jnp.zeros((pad,), jnp.int32)])
    dst_p = jnp.concatenate(
        [dst_flat[:E], jnp.full((pad,), N, jnp.int32),
         dst_flat[E:], jnp.full((pad,), N, jnp.int32)])
    idx_packed = jnp.stack([src_p.reshape(-1, CHUNK),
                            dst_p.reshape(-1, CHUNK)], axis=1)
    zfeat = jnp.zeros((RPT, D), f32)

    cnt = _sc_count(idx_packed, zfeat)
    y1, z1 = _stage_proj(x_all, wp, bp, wn1, ws1, bs1)
    s1 = _sc_segsum(y1.reshape(NC * N, D), idx_packed, zfeat)
    y2, z2 = _stage_mid(z1, s1, cnt, wn2, ws2, bs2)
    s2 = _sc_segsum(y2.reshape(NC * N, D), idx_packed, zfeat)
    out = _stage_fin(z2, s2, cnt)
    return out[0], out[1]


# speedup vs baseline: 1.1846x; 1.1846x over previous
import functools

import jax
import jax.numpy as jnp
from jax import lax
from jax.experimental import pallas as pl
from jax.experimental.pallas import tpu as pltpu
from jax.experimental.pallas import tpu_sc as plsc

N = 10000
E = 320000
D = 128
NC = 2
NS = 16
RPT = 624
TAIL = N - NS * RPT
CHUNK = 80
NCHUNK = -(-(E // NS) // CHUNK)
EPT = NCHUNK * CHUNK
EPAD = NS * EPT
NPAD = N + 16
BM = 2000


def _mmT(a, w):
    return lax.dot_general(a, w, (((1,), (1,)), ((), ())),
                           preferred_element_type=jnp.float32)


def _proj_body(x_ref, wp_ref, bp_ref, wn_ref, ws_ref, bs_ref, y_ref, z_ref):
    x = x_ref[0]
    h = jnp.maximum(_mmT(x, wp_ref[0]) + bp_ref[0, 0], 0.0)
    h = jnp.where(jnp.isnan(h), 0.0, h)
    y_ref[0] = _mmT(h, wn_ref[0])
    z_ref[0] = _mmT(h, ws_ref[0]) + bs_ref[0, 0]


def _mid_body(z_ref, s_ref, c_ref, wn_ref, ws_ref, bs_ref, y_ref, z2_ref):
    cnt = jnp.maximum(c_ref[0][:, 0:1], 1.0)
    h = jnp.maximum(z_ref[0] + s_ref[0] / cnt, 0.0)
    y_ref[0] = _mmT(h, wn_ref[0])
    z2_ref[0] = _mmT(h, ws_ref[0]) + bs_ref[0, 0]


def _fin_body(z_ref, s_ref, c_ref, o_ref):
    cnt = jnp.maximum(c_ref[0][:, 0:1], 1.0)
    o_ref[0] = jnp.maximum(z_ref[0] + s_ref[0] / cnt, 0.0)


def _wspec():
    return pl.BlockSpec((1, D, D), lambda t, r: (t, 0, 0))


def _bspec():
    return pl.BlockSpec((1, 1, D), lambda t, r: (t, 0, 0))


def _nspec():
    return pl.BlockSpec((1, BM, D), lambda t, r: (t, r, 0))


def _stage_proj(x_all, wp, bp, wn, ws, bs):
    return pl.pallas_call(
        _proj_body,
        grid=(2, N // BM),
        in_specs=[_nspec(), _wspec(), _bspec(), _wspec(), _wspec(), _bspec()],
        out_specs=[_nspec(), _nspec()],
        out_shape=[jax.ShapeDtypeStruct((2, N, D), jnp.float32)] * 2,
    )(x_all, wp, bp, wn, ws, bs)


def _stage_mid(z1, s1, c1, wn, ws, bs):
    return pl.pallas_call(
        _mid_body,
        grid=(2, N // BM),
        in_specs=[_nspec(), _nspec(), _nspec(), _wspec(), _wspec(), _bspec()],
        out_specs=[_nspec(), _nspec()],
        out_shape=[jax.ShapeDtypeStruct((2, N, D), jnp.float32)] * 2,
    )(z1, s1, c1, wn, ws, bs)


def _stage_fin(z2, s2, c2):
    return pl.pallas_call(
        _fin_body,
        grid=(2, N // BM),
        in_specs=[_nspec(), _nspec(), _nspec()],
        out_specs=_nspec(),
        out_shape=jax.ShapeDtypeStruct((2, N, D), jnp.float32),
    )(z2, s2, c2)


def _sc_mesh():
    return plsc.VectorSubcoreMesh(core_axis_name="c", subcore_axis_name="s",
                                  num_cores=NC, num_subcores=NS)


def _zero_phase(zf_hbm, acc, s):
    pltpu.sync_copy(zf_hbm, acc.at[pl.ds(s * RPT, RPT)])

    @pl.when(s == NS - 1)
    def _zero_tail():
        pltpu.sync_copy(zf_hbm.at[pl.ds(0, TAIL)],
                        acc.at[pl.ds(NS * RPT, TAIL)])


def _out_phase(acc, out, c, s):
    dts = 1 - c
    pltpu.sync_copy(acc.at[pl.ds(s * RPT, RPT)],
                    out.at[dts, pl.ds(s * RPT, RPT)])

    @pl.when(s == NS - 1)
    def _out_tail():
        pltpu.sync_copy(acc.at[pl.ds(NS * RPT, TAIL)],
                        out.at[dts, pl.ds(NS * RPT, TAIL)])


NBUF = 4


def _sc_segsum(y_flat, idx_packed, zfeat):
    scratch = (
        [pltpu.VMEM((2, CHUNK), jnp.int32)] * NBUF
        + [pltpu.VMEM((CHUNK, D), jnp.float32)] * NBUF
        + [pltpu.VMEM_SHARED((NPAD, D), jnp.float32)]
        + [pltpu.SemaphoreType.DMA] * (3 * NBUF)
    )

    @functools.partial(
        pl.kernel,
        out_type=jax.ShapeDtypeStruct((NC, N, D), jnp.float32),
        mesh=_sc_mesh(),
        scratch_types=scratch,
    )
    def body(y_hbm, idx_hbm, zf_hbm, s_out, *scr):
        ibuf = scr[0:NBUF]
        rows = scr[NBUF:2 * NBUF]
        acc = scr[2 * NBUF]
        isem = scr[2 * NBUF + 1:][0:NBUF]
        gsem = scr[2 * NBUF + 1:][NBUF:2 * NBUF]
        ssem = scr[2 * NBUF + 1:][2 * NBUF:3 * NBUF]

        c = lax.axis_index("c")
        s = lax.axis_index("s")
        _zero_phase(zf_hbm, acc, s)
        plsc.subcore_barrier()

        cbase = (c * NS + s) * NCHUNK

        def fire_idx(j, b):
            pltpu.async_copy(idx_hbm.at[cbase + j], ibuf[b], isem[b])

        def fire_gather(b):
            pltpu.make_async_copy(idx_hbm.at[cbase], ibuf[b], isem[b]).wait()
            pltpu.async_copy(y_hbm.at[ibuf[b].at[0]], rows[b], gsem[b])

        def fire_scatter(b):
            pltpu.make_async_copy(y_hbm.at[ibuf[b].at[0]], rows[b],
                                  gsem[b]).wait()
            pltpu.async_copy(rows[b], acc.at[ibuf[b].at[1]], ssem[b],
                             add=True)

        def wait_scatter(b):
            pltpu.make_async_copy(rows[b], acc.at[ibuf[b].at[1]],
                                  ssem[b]).wait()

        def macro(k, carry):
            for u in range(NBUF):
                t = k * NBUF + u
                pl.when((t >= NBUF) & (t - NBUF < NCHUNK))(
                    lambda b=u: wait_scatter(b))
                pl.when(t < NCHUNK)(lambda b=u, tt=t: fire_idx(tt, b))
                pl.when((t >= 1) & (t - 1 < NCHUNK))(
                    lambda b=(u - 1) % NBUF: fire_gather(b))
                pl.when((t >= 3) & (t - 3 < NCHUNK))(
                    lambda b=(u - 3) % NBUF: fire_scatter(b))
            return carry

        lax.fori_loop(0, (NCHUNK + NBUF - 1) // NBUF + 1, macro, 0)
        plsc.subcore_barrier()
        _out_phase(acc, s_out, c, s)

    return body(y_flat, idx_packed, zfeat)


def _sc_count(idx_packed, zfeat):
    @functools.partial(
        pl.kernel,
        out_type=jax.ShapeDtypeStruct((NC, N, D), jnp.float32),
        mesh=_sc_mesh(),
        scratch_types=(
            [pltpu.VMEM((2, CHUNK), jnp.int32)] * NBUF
            + [pltpu.VMEM((CHUNK, D), jnp.float32),
               pltpu.VMEM_SHARED((NPAD, D), jnp.float32)]
            + [pltpu.SemaphoreType.DMA] * (2 * NBUF)
        ),
    )
    def body(idx_hbm, zf_hbm, c_out, *scr):
        ibuf = scr[0:NBUF]
        ones = scr[NBUF]
        acc = scr[NBUF + 1]
        isem = scr[NBUF + 2:][0:NBUF]
        ssem = scr[NBUF + 2:][NBUF:2 * NBUF]

        c = lax.axis_index("c")
        s = lax.axis_index("s")
        _zero_phase(zf_hbm, acc, s)
        one16 = jnp.full((16,), 1.0, jnp.float32)

        def fill(i, carry):
            for g in range(D // 16):
                ones[i, pl.ds(g * 16, 16)] = one16
            return carry

        lax.fori_loop(0, CHUNK, fill, 0)
        plsc.subcore_barrier()

        cbase = (c * NS + s) * NCHUNK

        def fire_idx(j, b):
            pltpu.async_copy(idx_hbm.at[cbase + j], ibuf[b], isem[b])

        def fire_scatter(b):
            pltpu.make_async_copy(idx_hbm.at[cbase], ibuf[b], isem[b]).wait()
            pltpu.async_copy(ones, acc.at[ibuf[b].at[1]], ssem[b], add=True)

        def wait_scatter(b):
            pltpu.make_async_copy(ones, acc.at[ibuf[b].at[1]],
                                  ssem[b]).wait()

        def macro(k, carry):
            for u in range(NBUF):
                t = k * NBUF + u
                pl.when((t >= NBUF) & (t - NBUF < NCHUNK))(
                    lambda b=u: wait_scatter(b))
                pl.when(t < NCHUNK)(lambda b=u, tt=t: fire_idx(tt, b))
                pl.when((t >= 1) & (t - 1 < NCHUNK))(
                    lambda b=(u - 1) % NBUF: fire_scatter(b))
            return carry

        lax.fori_loop(0, (NCHUNK + NBUF - 1) // NBUF + 1, macro, 0)
        plsc.subcore_barrier()
        _out_phase(acc, c_out, c, s)

    return body(idx_packed, zfeat)


def kernel(x_user, x_item, edge_index_user_to_item, edge_index_item_rev_user,
           params):
    f32 = jnp.float32
    p = params
    x_all = jnp.stack([x_user, x_item]).astype(f32)

    wp = jnp.stack([p["proj"]["user"]["W"], p["proj"]["item"]["W"]])
    bp = jnp.stack([p["proj"]["user"]["b"], p["proj"]["item"]["b"]])[:, None, :]
    c1u = p["conv1"]["user__to__item"]
    c1i = p["conv1"]["item__rev__user"]
    c2u = p["conv2"]["user__to__item"]
    c2i = p["conv2"]["item__rev__user"]
    wn1 = jnp.stack([c1u["Wn"], c1i["Wn"]])
    ws1 = jnp.stack([c1i["Ws"], c1u["Ws"]])
    bs1 = jnp.stack([c1i["bs"], c1u["bs"]])[:, None, :]
    wn2 = jnp.stack([c2u["Wn"], c2i["Wn"]])
    ws2 = jnp.stack([c2i["Ws"], c2u["Ws"]])
    bs2 = jnp.stack([c2i["bs"], c2u["bs"]])[:, None, :]

    src_flat = jnp.concatenate([edge_index_user_to_item[0].astype(jnp.int32),
                                edge_index_item_rev_user[0].astype(jnp.int32)
                                + N])
    dst_flat = jnp.concatenate([edge_index_user_to_item[1].astype(jnp.int32),
                                edge_index_item_rev_user[1].astype(jnp.int32)])
    pad = EPAD - E
    src_p = jnp.concatenate(
        [src_flat[:E], jnp.zeros((pad,), jnp.int32),
         src_flat[E:], jnp.zeros((pad,), jnp.int32)])
    dst_p = jnp.concatenate(
        [dst_flat[:E], jnp.full((pad,), N, jnp.int32),
         dst_flat[E:], jnp.full((pad,), N, jnp.int32)])
    idx_packed = jnp.stack([src_p.reshape(-1, CHUNK),
                            dst_p.reshape(-1, CHUNK)], axis=1)
    zfeat = jnp.zeros((RPT, D), f32)

    cnt = _sc_count(idx_packed, zfeat)
    y1, z1 = _stage_proj(x_all, wp, bp, wn1, ws1, bs1)
    s1 = _sc_segsum(y1.reshape(NC * N, D), idx_packed, zfeat)
    y2, z2 = _stage_mid(z1, s1, cnt, wn2, ws2, bs2)
    s2 = _sc_segsum(y2.reshape(NC * N, D), idx_packed, zfeat)
    out = _stage_fin(z2, s2, cnt)
    return out[0], out[1]


# speedup vs baseline: 1.1979x; 1.0113x over previous
import functools

import jax
import jax.numpy as jnp
from jax import lax
from jax.experimental import pallas as pl
from jax.experimental.pallas import tpu as pltpu
from jax.experimental.pallas import tpu_sc as plsc

N = 10000
E = 320000
D = 128
NC = 2
NS = 16
RPT = 624
TAIL = N - NS * RPT
CHUNK = 80
NCHUNK = -(-(E // NS) // CHUNK)
EPT = NCHUNK * CHUNK
EPAD = NS * EPT
NPAD = N + 16
BM = 2000


def _mmT(a, w):
    return lax.dot_general(a, w, (((1,), (1,)), ((), ())),
                           preferred_element_type=jnp.float32)


def _proj_body(x_ref, wp_ref, bp_ref, wn_ref, ws_ref, bs_ref, y_ref, z_ref):
    x = x_ref[0]
    h = jnp.maximum(_mmT(x, wp_ref[0]) + bp_ref[0, 0], 0.0)
    h = jnp.where(jnp.isnan(h), 0.0, h)
    y_ref[0] = _mmT(h, wn_ref[0])
    z_ref[0] = _mmT(h, ws_ref[0]) + bs_ref[0, 0]


def _mid_body(z_ref, s_ref, c_ref, wn_ref, ws_ref, bs_ref, y_ref, z2_ref):
    cnt = jnp.maximum(c_ref[0][:, 0:1], 1.0)
    h = jnp.maximum(z_ref[0] + s_ref[0] / cnt, 0.0)
    y_ref[0] = _mmT(h, wn_ref[0])
    z2_ref[0] = _mmT(h, ws_ref[0]) + bs_ref[0, 0]


def _fin_body(z_ref, s_ref, c_ref, o_ref):
    cnt = jnp.maximum(c_ref[0][:, 0:1], 1.0)
    o_ref[0] = jnp.maximum(z_ref[0] + s_ref[0] / cnt, 0.0)


def _wspec():
    return pl.BlockSpec((1, D, D), lambda t, r: (t, 0, 0))


def _bspec():
    return pl.BlockSpec((1, 1, D), lambda t, r: (t, 0, 0))


def _nspec():
    return pl.BlockSpec((1, BM, D), lambda t, r: (t, r, 0))


def _stage_proj(x_all, wp, bp, wn, ws, bs):
    return pl.pallas_call(
        _proj_body,
        grid=(2, N // BM),
        in_specs=[_nspec(), _wspec(), _bspec(), _wspec(), _wspec(), _bspec()],
        out_specs=[_nspec(), _nspec()],
        out_shape=[jax.ShapeDtypeStruct((2, N, D), jnp.float32)] * 2,
    )(x_all, wp, bp, wn, ws, bs)


def _stage_mid(z1, s1, c1, wn, ws, bs):
    return pl.pallas_call(
        _mid_body,
        grid=(2, N // BM),
        in_specs=[_nspec(), _nspec(), _nspec(), _wspec(), _wspec(), _bspec()],
        out_specs=[_nspec(), _nspec()],
        out_shape=[jax.ShapeDtypeStruct((2, N, D), jnp.float32)] * 2,
    )(z1, s1, c1, wn, ws, bs)


def _stage_fin(z2, s2, c2):
    return pl.pallas_call(
        _fin_body,
        grid=(2, N // BM),
        in_specs=[_nspec(), _nspec(), _nspec()],
        out_specs=_nspec(),
        out_shape=jax.ShapeDtypeStruct((2, N, D), jnp.float32),
    )(z2, s2, c2)


def _sc_mesh():
    return plsc.VectorSubcoreMesh(core_axis_name="c", subcore_axis_name="s",
                                  num_cores=NC, num_subcores=NS)


def _zero_phase(zf_hbm, acc, s):
    pltpu.sync_copy(zf_hbm, acc.at[pl.ds(s * RPT, RPT)])

    @pl.when(s == NS - 1)
    def _zero_tail():
        pltpu.sync_copy(zf_hbm.at[pl.ds(0, TAIL)],
                        acc.at[pl.ds(NS * RPT, TAIL)])


def _out_phase(acc, out, c, s):
    dts = 1 - c
    pltpu.sync_copy(acc.at[pl.ds(s * RPT, RPT)],
                    out.at[dts, pl.ds(s * RPT, RPT)])

    @pl.when(s == NS - 1)
    def _out_tail():
        pltpu.sync_copy(acc.at[pl.ds(NS * RPT, TAIL)],
                        out.at[dts, pl.ds(NS * RPT, TAIL)])


NBUF = 4


def _sc_segsum(y_flat, idx_packed, zfeat, with_count=False):
    scratch = (
        [pltpu.VMEM((2, CHUNK), jnp.int32)] * NBUF
        + [pltpu.VMEM((CHUNK, D), jnp.float32)] * NBUF
        + [pltpu.VMEM_SHARED((NPAD, D), jnp.float32)]
        + [pltpu.SemaphoreType.DMA] * (3 * NBUF)
    )

    out_type = jax.ShapeDtypeStruct((NC, N, D), jnp.float32)
    if with_count:
        out_type = (out_type, jax.ShapeDtypeStruct((NC, N, D), jnp.float32))

    @functools.partial(
        pl.kernel,
        out_type=out_type,
        mesh=_sc_mesh(),
        scratch_types=scratch,
    )
    def body(y_hbm, idx_hbm, zf_hbm, *outs_scr):
        if with_count:
            s_out, c_out = outs_scr[0], outs_scr[1]
            scr = outs_scr[2:]
        else:
            s_out = outs_scr[0]
            scr = outs_scr[1:]
        ibuf = scr[0:NBUF]
        rows = scr[NBUF:2 * NBUF]
        acc = scr[2 * NBUF]
        isem = scr[2 * NBUF + 1:][0:NBUF]
        gsem = scr[2 * NBUF + 1:][NBUF:2 * NBUF]
        ssem = scr[2 * NBUF + 1:][2 * NBUF:3 * NBUF]

        c = lax.axis_index("c")
        s = lax.axis_index("s")
        _zero_phase(zf_hbm, acc, s)

        cbase = (c * NS + s) * NCHUNK

        def fire_idx(j, b):
            pltpu.async_copy(idx_hbm.at[cbase + j], ibuf[b], isem[b])

        if with_count:
            one16 = jnp.full((16,), 1.0, jnp.float32)

            def fill(i, carry):
                for g in range(D // 16):
                    rows[0][i, pl.ds(g * 16, 16)] = one16
                return carry

            lax.fori_loop(0, CHUNK, fill, 0)
            plsc.subcore_barrier()

            def cfire_scatter(b):
                pltpu.make_async_copy(idx_hbm.at[cbase], ibuf[b],
                                      isem[b]).wait()
                pltpu.async_copy(rows[0], acc.at[ibuf[b].at[1]], ssem[b],
                                 add=True)

            def cwait_scatter(b):
                pltpu.make_async_copy(rows[0], acc.at[ibuf[b].at[1]],
                                      ssem[b]).wait()

            def cmacro(k, carry):
                for u in range(NBUF):
                    t = k * NBUF + u
                    pl.when((t >= NBUF) & (t - NBUF < NCHUNK))(
                        lambda b=u: cwait_scatter(b))
                    pl.when(t < NCHUNK)(lambda b=u, tt=t: fire_idx(tt, b))
                    pl.when((t >= 1) & (t - 1 < NCHUNK))(
                        lambda b=(u - 1) % NBUF: cfire_scatter(b))
                return carry

            lax.fori_loop(0, (NCHUNK + NBUF - 1) // NBUF + 1, cmacro, 0)
            plsc.subcore_barrier()
            _out_phase(acc, c_out, c, s)
            _zero_phase(zf_hbm, acc, s)

        plsc.subcore_barrier()

        def fire_gather(b):
            pltpu.make_async_copy(idx_hbm.at[cbase], ibuf[b], isem[b]).wait()
            pltpu.async_copy(y_hbm.at[ibuf[b].at[0]], rows[b], gsem[b])

        def fire_scatter(b):
            pltpu.make_async_copy(y_hbm.at[ibuf[b].at[0]], rows[b],
                                  gsem[b]).wait()
            pltpu.async_copy(rows[b], acc.at[ibuf[b].at[1]], ssem[b],
                             add=True)

        def wait_scatter(b):
            pltpu.make_async_copy(rows[b], acc.at[ibuf[b].at[1]],
                                  ssem[b]).wait()

        def macro(k, carry):
            for u in range(NBUF):
                t = k * NBUF + u
                pl.when((t >= NBUF) & (t - NBUF < NCHUNK))(
                    lambda b=u: wait_scatter(b))
                pl.when(t < NCHUNK)(lambda b=u, tt=t: fire_idx(tt, b))
                pl.when((t >= 1) & (t - 1 < NCHUNK))(
                    lambda b=(u - 1) % NBUF: fire_gather(b))
                pl.when((t >= 3) & (t - 3 < NCHUNK))(
                    lambda b=(u - 3) % NBUF: fire_scatter(b))
            return carry

        lax.fori_loop(0, (NCHUNK + NBUF - 1) // NBUF + 1, macro, 0)
        plsc.subcore_barrier()
        _out_phase(acc, s_out, c, s)

    return body(y_flat, idx_packed, zfeat)


def _sc_count_unused(idx_packed, zfeat):
    @functools.partial(
        pl.kernel,
        out_type=jax.ShapeDtypeStruct((NC, N, D), jnp.float32),
        mesh=_sc_mesh(),
        scratch_types=(
            [pltpu.VMEM((2, CHUNK), jnp.int32)] * NBUF
            + [pltpu.VMEM((CHUNK, D), jnp.float32),
               pltpu.VMEM_SHARED((NPAD, D), jnp.float32)]
            + [pltpu.SemaphoreType.DMA] * (2 * NBUF)
        ),
    )
    def body(idx_hbm, zf_hbm, c_out, *scr):
        ibuf = scr[0:NBUF]
        ones = scr[NBUF]
        acc = scr[NBUF + 1]
        isem = scr[NBUF + 2:][0:NBUF]
        ssem = scr[NBUF + 2:][NBUF:2 * NBUF]

        c = lax.axis_index("c")
        s = lax.axis_index("s")
        _zero_phase(zf_hbm, acc, s)
        one16 = jnp.full((16,), 1.0, jnp.float32)

        def fill(i, carry):
            for g in range(D // 16):
                ones[i, pl.ds(g * 16, 16)] = one16
            return carry

        lax.fori_loop(0, CHUNK, fill, 0)
        plsc.subcore_barrier()

        cbase = (c * NS + s) * NCHUNK

        def fire_idx(j, b):
            pltpu.async_copy(idx_hbm.at[cbase + j], ibuf[b], isem[b])

        def fire_scatter(b):
            pltpu.make_async_copy(idx_hbm.at[cbase], ibuf[b], isem[b]).wait()
            pltpu.async_copy(ones, acc.at[ibuf[b].at[1]], ssem[b], add=True)

        def wait_scatter(b):
            pltpu.make_async_copy(ones, acc.at[ibuf[b].at[1]],
                                  ssem[b]).wait()

        def macro(k, carry):
            for u in range(NBUF):
                t = k * NBUF + u
                pl.when((t >= NBUF) & (t - NBUF < NCHUNK))(
                    lambda b=u: wait_scatter(b))
                pl.when(t < NCHUNK)(lambda b=u, tt=t: fire_idx(tt, b))
                pl.when((t >= 1) & (t - 1 < NCHUNK))(
                    lambda b=(u - 1) % NBUF: fire_scatter(b))
            return carry

        lax.fori_loop(0, (NCHUNK + NBUF - 1) // NBUF + 1, macro, 0)
        plsc.subcore_barrier()
        _out_phase(acc, c_out, c, s)

    return body(idx_packed, zfeat)


def kernel(x_user, x_item, edge_index_user_to_item, edge_index_item_rev_user,
           params):
    f32 = jnp.float32
    p = params
    x_all = jnp.stack([x_user, x_item]).astype(f32)

    wp = jnp.stack([p["proj"]["user"]["W"], p["proj"]["item"]["W"]])
    bp = jnp.stack([p["proj"]["user"]["b"], p["proj"]["item"]["b"]])[:, None, :]
    c1u = p["conv1"]["user__to__item"]
    c1i = p["conv1"]["item__rev__user"]
    c2u = p["conv2"]["user__to__item"]
    c2i = p["conv2"]["item__rev__user"]
    wn1 = jnp.stack([c1u["Wn"], c1i["Wn"]])
    ws1 = jnp.stack([c1i["Ws"], c1u["Ws"]])
    bs1 = jnp.stack([c1i["bs"], c1u["bs"]])[:, None, :]
    wn2 = jnp.stack([c2u["Wn"], c2i["Wn"]])
    ws2 = jnp.stack([c2i["Ws"], c2u["Ws"]])
    bs2 = jnp.stack([c2i["bs"], c2u["bs"]])[:, None, :]

    src_flat = jnp.concatenate([edge_index_user_to_item[0].astype(jnp.int32),
                                edge_index_item_rev_user[0].astype(jnp.int32)
                                + N])
    dst_flat = jnp.concatenate([edge_index_user_to_item[1].astype(jnp.int32),
                                edge_index_item_rev_user[1].astype(jnp.int32)])
    pad = EPAD - E
    src_p = jnp.concatenate(
        [src_flat[:E], jnp.zeros((pad,), jnp.int32),
         src_flat[E:], jnp.zeros((pad,), jnp.int32)])
    dst_p = jnp.concatenate(
        [dst_flat[:E], jnp.full((pad,), N, jnp.int32),
         dst_flat[E:], jnp.full((pad,), N, jnp.int32)])
    idx_packed = jnp.stack([src_p.reshape(-1, CHUNK),
                            dst_p.reshape(-1, CHUNK)], axis=1)
    zfeat = jnp.zeros((RPT, D), f32)

    y1, z1 = _stage_proj(x_all, wp, bp, wn1, ws1, bs1)
    s1, cnt = _sc_segsum(y1.reshape(NC * N, D), idx_packed, zfeat,
                         with_count=True)
    y2, z2 = _stage_mid(z1, s1, cnt, wn2, ws2, bs2)
    s2 = _sc_segsum(y2.reshape(NC * N, D), idx_packed, zfeat)
    out = _stage_fin(z2, s2, cnt)
    return out[0], out[1]
